# baseline (speedup 1.0000x reference)
import functools
import math

import jax
import jax.numpy as jnp
from jax import lax
from jax.experimental import pallas as pl
from jax.experimental.pallas import tpu as pltpu
from jax.experimental.pallas import tpu_sc as plsc


def _router_body(C, EC, chunk, x_ref, gw_ref, route_ref, meta_ref, oh_ref):
    x = x_ref[:]
    gw = gw_ref[:]
    N = x.shape[0]
    E = gw.shape[0]
    logits = lax.dot_general(x, gw, (((1,), (1,)), ((), ())),
                             preferred_element_type=jnp.float32)
    m = jnp.max(logits, axis=1, keepdims=True)
    p = jnp.exp(logits - m)
    gates = p / jnp.sum(p, axis=1, keepdims=True)
    gate = jnp.max(gates, axis=1, keepdims=True)
    iota_e = lax.broadcasted_iota(jnp.int32, (N, E), 1).astype(jnp.float32)
    e_idx = jnp.min(jnp.where(gates >= gate, iota_e, jnp.float32(E)),
                    axis=1, keepdims=True)
    oh_ref[:] = (iota_e == e_idx).astype(jnp.float32)
    route_ref[:, 0:1] = e_idx
    route_ref[:, 2:3] = gate

    def body(i, _):
        base = i * chunk
        r_i = lax.broadcasted_iota(jnp.int32, (chunk, N), 0) + base
        c_i = lax.broadcasted_iota(jnp.int32, (chunk, N), 1)
        tri = (c_i <= r_i).astype(jnp.float32)
        csum = lax.dot_general(tri, oh_ref[:], (((1,), (0,)), ((), ())),
                               preferred_element_type=jnp.float32)
        oh_c = oh_ref[pl.ds(base, chunk), :]
        pos = jnp.sum(csum * oh_c, axis=1, keepdims=True) - 1.0
        route_ref[pl.ds(base, chunk), 1:2] = pos
        return 0

    lax.fori_loop(0, N // chunk, body, 0)
    e_col = route_ref[:, 0:1]
    p_col = route_ref[:, 1:2]
    slot = jnp.where(p_col < jnp.float32(C), e_col * jnp.float32(C) + p_col,
                     jnp.float32(EC))
    n_i = lax.broadcasted_iota(jnp.int32, (N, 1), 0)
    n_hi = (n_i // 16).astype(jnp.float32)
    n_lo = (n_i % 16).astype(jnp.float32)
    n2 = jnp.concatenate([n_hi, n_lo], axis=1)
    sc = 512
    for c in range(EC // sc):
        iota_s = (lax.broadcasted_iota(jnp.int32, (N, sc), 1)
                  + c * sc).astype(jnp.float32)
        S = (slot == iota_s).astype(jnp.float32)
        d2 = lax.dot_general(S, n2, (((0,), (0,)), ((), ())),
                             preferred_element_type=jnp.float32)
        meta_ref[c * sc:(c + 1) * sc, 0:1] = (
            d2[:, 0:1] * 16.0 + d2[:, 1:2])


def _sc_gather_body(EC, meta_hbm, x_hbm, xe_hbm, midx_v, idx_v, rows_v, sem):
    rows_per = EC // 32
    wid = lax.axis_index("s") * 2 + lax.axis_index("c")
    pltpu.sync_copy(meta_hbm.at[wid], midx_v)

    def cv(j, _):
        idx_v[pl.ds(j * 16, 16)] = lax.convert_element_type(
            midx_v[pl.ds(j * 16, 16)], jnp.int32)
        return 0

    lax.fori_loop(0, rows_per // 16, cv, 0)
    pltpu.async_copy(x_hbm.at[idx_v], rows_v, sem).wait()
    pltpu.sync_copy(rows_v, xe_hbm.at[pl.ds(wid * rows_per, rows_per)])


def _expert_body(C, xe_ref, route_ref, w1_ref, b1_ref, w2_ref, b2_ref,
                 out_ref):
    e = pl.program_id(0)
    N = route_ref.shape[0]

    @pl.when(e == 0)
    def _():
        out_ref[:] = jnp.zeros_like(out_ref)

    ef = lax.convert_element_type(e, jnp.float32)
    ecol = route_ref[:, 0:1]
    pcol = route_ref[:, 1:2]
    gcol = route_ref[:, 2:3]
    keep = (ecol == ef) & (pcol < jnp.float32(C))
    iota_c = lax.broadcasted_iota(jnp.int32, (N, C), 1).astype(jnp.float32)
    P = jnp.where(keep & (pcol == iota_c), 1.0, 0.0)
    h = lax.dot_general(xe_ref[:], w1_ref[0], (((1,), (1,)), ((), ())),
                        preferred_element_type=jnp.float32) + b1_ref[0]
    h = 0.5 * h * (1.0 + lax.erf(h * 0.7071067811865476))
    o = lax.dot_general(h, w2_ref[0], (((1,), (1,)), ((), ())),
                        preferred_element_type=jnp.float32) + b2_ref[0]
    out_ref[:] += lax.dot_general(P * gcol, o, (((1,), (0,)), ((), ())),
                                  preferred_element_type=jnp.float32)


def kernel(hidden_states, gate_w, W1, b1, W2, b2):
    Bs, Ts, D = hidden_states.shape
    N = Bs * Ts
    E, H = W1.shape[0], W1.shape[1]
    C = math.ceil(2.0 * N / E)
    EC = E * C
    flat = hidden_states.reshape(N, D)

    route, meta = pl.pallas_call(
        functools.partial(_router_body, C, EC, 128),
        out_shape=(jax.ShapeDtypeStruct((N, 128), jnp.float32),
                   jax.ShapeDtypeStruct((EC, 1), jnp.float32)),
        scratch_shapes=[pltpu.VMEM((N, E), jnp.float32)],
    )(flat, gate_w)

    mesh = plsc.VectorSubcoreMesh(core_axis_name="c", subcore_axis_name="s")
    xe = pl.kernel(
        functools.partial(_sc_gather_body, EC),
        mesh=mesh,
        out_type=jax.ShapeDtypeStruct((EC, D), jnp.float32),
        scratch_types=[
            pltpu.VMEM((EC // 32,), jnp.float32),
            pltpu.VMEM((EC // 32,), jnp.int32),
            pltpu.VMEM((EC // 32, D), jnp.float32),
            pltpu.SemaphoreType.DMA,
        ],
    )(meta.reshape(32, EC // 32), flat)

    final = pl.pallas_call(
        functools.partial(_expert_body, C),
        grid=(E,),
        in_specs=[
            pl.BlockSpec((C, D), lambda e: (e, 0)),
            pl.BlockSpec((N, 128), lambda e: (0, 0)),
            pl.BlockSpec((1, H, D), lambda e: (e, 0, 0)),
            pl.BlockSpec((1, 1, H), lambda e: (e, 0, 0)),
            pl.BlockSpec((1, D, H), lambda e: (e, 0, 0)),
            pl.BlockSpec((1, 1, D), lambda e: (e, 0, 0)),
        ],
        out_specs=pl.BlockSpec((N, D), lambda e: (0, 0)),
        out_shape=jax.ShapeDtypeStruct((N, D), jnp.float32),
        scratch_shapes=[],
    )(xe, route, W1, b1.reshape(E, 1, H), W2, b2.reshape(E, 1, D))

    aux_loss = jnp.asarray(0.0, dtype=jnp.float32)
    return final.reshape(Bs, Ts, D), aux_loss


# speedup vs baseline: 1.2300x; 1.2300x over previous
import functools
import math

import jax
import jax.numpy as jnp
from jax import lax
from jax.experimental import pallas as pl
from jax.experimental.pallas import tpu as pltpu


def _route(chunk, x_ref, gw_ref, route_ref, oh_ref):
    x = x_ref[:]
    gw = gw_ref[:]
    N = x.shape[0]
    E = gw.shape[0]
    logits = lax.dot_general(x, gw, (((1,), (1,)), ((), ())),
                             preferred_element_type=jnp.float32)
    m = jnp.max(logits, axis=1, keepdims=True)
    p = jnp.exp(logits - m)
    gates = p / jnp.sum(p, axis=1, keepdims=True)
    gate = jnp.max(gates, axis=1, keepdims=True)
    iota_e = lax.broadcasted_iota(jnp.int32, (N, E), 1).astype(jnp.float32)
    e_idx = jnp.min(jnp.where(gates >= gate, iota_e, jnp.float32(E)),
                    axis=1, keepdims=True)
    oh_ref[:] = (iota_e == e_idx).astype(jnp.float32)
    route_ref[:, 0:1] = e_idx
    route_ref[:, 2:3] = gate

    def body(i, _):
        base = i * chunk
        r_i = lax.broadcasted_iota(jnp.int32, (chunk, N), 0) + base
        c_i = lax.broadcasted_iota(jnp.int32, (chunk, N), 1)
        tri = (c_i <= r_i).astype(jnp.float32)
        csum = lax.dot_general(tri, oh_ref[:], (((1,), (0,)), ((), ())),
                               preferred_element_type=jnp.float32)
        oh_c = oh_ref[pl.ds(base, chunk), :]
        pos = jnp.sum(csum * oh_c, axis=1, keepdims=True) - 1.0
        route_ref[pl.ds(base, chunk), 1:2] = pos
        return 0

    lax.fori_loop(0, N // chunk, body, 0)


def _body(C, chunk, x_ref, gw_ref, w1_ref, b1_ref, w2_ref, b2_ref, out_ref,
          route_ref, oh_ref):
    e = pl.program_id(0)
    N = x_ref.shape[0]

    @pl.when(e == 0)
    def _():
        _route(chunk, x_ref, gw_ref, route_ref, oh_ref)
        out_ref[:] = jnp.zeros_like(out_ref)

    ef = lax.convert_element_type(e, jnp.float32)
    ecol = route_ref[:, 0:1]
    pcol = route_ref[:, 1:2]
    gcol = route_ref[:, 2:3]
    keep = (ecol == ef) & (pcol < jnp.float32(C))
    iota_c = lax.broadcasted_iota(jnp.int32, (N, C), 1).astype(jnp.float32)
    P = jnp.where(keep & (pcol == iota_c), 1.0, 0.0)
    xe = lax.dot_general(P, x_ref[:], (((0,), (0,)), ((), ())),
                         preferred_element_type=jnp.float32)
    h = lax.dot_general(xe, w1_ref[0], (((1,), (1,)), ((), ())),
                        preferred_element_type=jnp.float32) + b1_ref[0]
    h = 0.5 * h * (1.0 + lax.erf(h * 0.7071067811865476))
    o = lax.dot_general(h, w2_ref[0], (((1,), (1,)), ((), ())),
                        preferred_element_type=jnp.float32) + b2_ref[0]
    out_ref[:] += lax.dot_general(P * gcol, o, (((1,), (0,)), ((), ())),
                                  preferred_element_type=jnp.float32)


def kernel(hidden_states, gate_w, W1, b1, W2, b2):
    Bs, Ts, D = hidden_states.shape
    N = Bs * Ts
    E, H = W1.shape[0], W1.shape[1]
    C = math.ceil(2.0 * N / E)
    flat = hidden_states.reshape(N, D)

    final = pl.pallas_call(
        functools.partial(_body, C, 128),
        grid=(E,),
        in_specs=[
            pl.BlockSpec((N, D), lambda e: (0, 0)),
            pl.BlockSpec((E, D), lambda e: (0, 0)),
            pl.BlockSpec((1, H, D), lambda e: (e, 0, 0)),
            pl.BlockSpec((1, 1, H), lambda e: (e, 0, 0)),
            pl.BlockSpec((1, D, H), lambda e: (e, 0, 0)),
            pl.BlockSpec((1, 1, D), lambda e: (e, 0, 0)),
        ],
        out_specs=pl.BlockSpec((N, D), lambda e: (0, 0)),
        out_shape=jax.ShapeDtypeStruct((N, D), jnp.float32),
        scratch_shapes=[
            pltpu.VMEM((N, 128), jnp.float32),
            pltpu.VMEM((N, E), jnp.float32),
        ],
    )(flat, gate_w, W1, b1.reshape(E, 1, H), W2, b2.reshape(E, 1, D))

    aux_loss = jnp.asarray(0.0, dtype=jnp.float32)
    return final.reshape(Bs, Ts, D), aux_loss
